# baseline (speedup 1.0000x reference)
import functools

import jax
import jax.numpy as jnp
from jax import lax
from jax.experimental import pallas as pl
from jax.experimental.pallas import tpu as pltpu
from jax.experimental.pallas import tpu_sc as plsc

N = 10000
D = 128
HID = 8
NCLS = 16
TBW = 16
N_PAD = 10240
N_PACK = N_PAD // 8
E = 320000
NC = 2
NS = 16
CHUNK = 128
CPT0 = 144
CPT1 = 16
NCHUNKS = NS * (CPT0 + CPT1)
E_PAD = NCHUNKS * CHUNK
ROWS_PT = N_PAD // NS

_mesh = plsc.VectorSubcoreMesh(core_axis_name="c", subcore_axis_name="s")


@functools.partial(
    pl.kernel,
    out_type=jax.ShapeDtypeStruct((NC, N_PAD, TBW), jnp.float32),
    mesh=_mesh,
    compiler_params=pltpu.CompilerParams(use_tc_tiling_on_sc=False),
    scratch_types=[
        pltpu.VMEM((CPT0, CHUNK), jnp.int32),
        pltpu.VMEM((CPT0, CHUNK), jnp.int32),
        [pltpu.VMEM((CHUNK, TBW), jnp.float32)] * 8,
        pltpu.VMEM_SHARED((N_PAD, TBW), jnp.float32),
        [pltpu.SemaphoreType.DMA] * 4,
    ],
)
def _edge_agg(table_hbm, edges_hbm, out_hbm,
              src_v, dst_v, rows, acc, sems):
    c = lax.axis_index("c")
    s = lax.axis_index("s")
    row0 = s * ROWS_PT
    @pl.loop(0, CHUNK)
    def _(i):
        rows[0][i] = jnp.zeros((TBW,), jnp.float32)

    for t in range(ROWS_PT // CHUNK):
        pltpu.sync_copy(rows[0], acc.at[pl.ds(row0 + t * CHUNK, CHUNK)])

    K = 4
    bufs = (rows[:K], rows[K:])
    gsem = (sems[0], sems[1])
    ssem = (sems[2], sems[3])

    def _fire_g(g, j):
        for k in range(K):
            pltpu.async_copy(table_hbm.at[src_v.at[j + k]], bufs[g][k],
                             gsem[g])

    def _drain_g(g):
        for k in range(K):
            pltpu.make_async_copy(table_hbm.at[src_v.at[0]], bufs[g][k],
                                  gsem[g]).wait()

    def _fire_s(g, j):
        for k in range(K):
            pltpu.async_copy(bufs[g][k], acc.at[dst_v.at[j + k]], ssem[g],
                             add=True)

    def _drain_s(g):
        for k in range(K):
            pltpu.make_async_copy(bufs[g][k], acc.at[dst_v.at[0]],
                                  ssem[g]).wait()

    def _run(base, cpt):
        pltpu.sync_copy(edges_hbm.at[0, pl.ds(base, cpt)],
                        src_v.at[pl.ds(0, cpt)])
        pltpu.sync_copy(edges_hbm.at[1, pl.ds(base, cpt)],
                        dst_v.at[pl.ds(0, cpt)])
        plsc.subcore_barrier()

        _fire_g(0, 0)

        @pl.loop(0, cpt - 2 * K, step=2 * K)
        def _(j):
            _fire_g(1, j + K)
            _drain_g(0)
            _fire_s(0, j)
            _drain_s(0)
            _fire_g(0, j + 2 * K)
            _drain_g(1)
            _fire_s(1, j + K)
            _drain_s(1)

        _fire_g(1, cpt - K)
        _drain_g(0)
        _fire_s(0, cpt - 2 * K)
        _drain_s(0)
        _drain_g(1)
        _fire_s(1, cpt - K)
        _drain_s(1)

    @pl.when(c == 0)
    def _():
        _run(s * CPT0, CPT0)

    @pl.when(c != 0)
    def _():
        _run(NS * CPT0 + s * CPT1, CPT1)

    plsc.subcore_barrier()
    pltpu.sync_copy(acc.at[pl.ds(row0, ROWS_PT)],
                    out_hbm.at[c, pl.ds(row0, ROWS_PT)])


BLK = 128
GRID = N_PACK // BLK


def _iota2(shape, dim):
    return lax.broadcasted_iota(jnp.int32, shape, dim)


def _pre_body(x_ref, wl_ref, wr_ref, b1_ref, t1_ref, r1_ref):
    xb = x_ref[...]
    wl = wl_ref[...]
    wr = wr_ref[...]
    t = jnp.concatenate(
        [jnp.dot(xb[:, i * D:(i + 1) * D], wl,
                 preferred_element_type=jnp.float32) for i in range(8)],
        axis=1)
    col = _iota2((BLK, 128), 1)
    t1_ref[...] = jnp.where(col % TBW == HID, t + 1.0, t)
    r = jnp.concatenate(
        [jnp.dot(xb[:, i * D:(i + 1) * D], wr,
                 preferred_element_type=jnp.float32) for i in range(8)],
        axis=1)
    r1_ref[...] = r + b1_ref[...]


_pre = pl.pallas_call(
    _pre_body,
    grid=(GRID,),
    in_specs=[pl.BlockSpec((BLK, 8 * D), lambda i: (i, 0)),
              pl.BlockSpec((D, TBW), lambda i: (0, 0)),
              pl.BlockSpec((D, HID), lambda i: (0, 0)),
              pl.BlockSpec((1, 64), lambda i: (0, 0))],
    out_specs=[pl.BlockSpec((BLK, 128), lambda i: (i, 0)),
               pl.BlockSpec((BLK, 64), lambda i: (i, 0))],
    out_shape=[jax.ShapeDtypeStruct((N_PACK, 128), jnp.float32),
               jax.ShapeDtypeStruct((N_PACK, 64), jnp.float32)],
)


def _mid_body(p_ref, r1_ref, wl_ref, wr_ref, b2_ref, t2_ref, r2_ref, den_ref):
    ssum = p_ref[0] + p_ref[1]
    row = _iota2((128, 64), 0)
    colc = _iota2((128, 64), 1)
    sel_sum = ((colc // HID) * TBW + colc % HID == row).astype(jnp.float32)
    sel_cnt = ((colc // HID) * TBW + HID == row).astype(jnp.float32)
    sums = jnp.dot(ssum, sel_sum, preferred_element_type=jnp.float32)
    dens = jnp.maximum(
        jnp.dot(ssum, sel_cnt, preferred_element_type=jnp.float32), 1.0)
    h = sums / dens + r1_ref[...]
    h = jnp.where(h > 0, h, jnp.exp(jnp.minimum(h, 0.0)) - 1.0)
    wl = wl_ref[...]
    wr = wr_ref[...]
    t2_ref[...] = jnp.concatenate(
        [jnp.dot(h[:, i * HID:(i + 1) * HID], wl,
                 preferred_element_type=jnp.float32) for i in range(8)],
        axis=1)
    r2 = jnp.concatenate(
        [jnp.dot(h[:, i * HID:(i + 1) * HID], wr,
                 preferred_element_type=jnp.float32) for i in range(8)],
        axis=1)
    r2_ref[...] = r2 + b2_ref[...]
    rowc = _iota2((128, 8), 0)
    col8 = _iota2((128, 8), 1)
    sel_den = (col8 * TBW + HID == rowc).astype(jnp.float32)
    den_ref[...] = jnp.maximum(
        jnp.dot(ssum, sel_den, preferred_element_type=jnp.float32), 1.0)


_mid = pl.pallas_call(
    _mid_body,
    grid=(GRID,),
    in_specs=[pl.BlockSpec((NC, BLK, 128), lambda i: (0, i, 0)),
              pl.BlockSpec((BLK, 64), lambda i: (i, 0)),
              pl.BlockSpec((HID, NCLS), lambda i: (0, 0)),
              pl.BlockSpec((HID, NCLS), lambda i: (0, 0)),
              pl.BlockSpec((1, 128), lambda i: (0, 0))],
    out_specs=[pl.BlockSpec((BLK, 128), lambda i: (i, 0)),
               pl.BlockSpec((BLK, 128), lambda i: (i, 0)),
               pl.BlockSpec((BLK, 8), lambda i: (i, 0))],
    out_shape=[jax.ShapeDtypeStruct((N_PACK, 128), jnp.float32),
               jax.ShapeDtypeStruct((N_PACK, 128), jnp.float32),
               jax.ShapeDtypeStruct((N_PACK, 8), jnp.float32)],
)


def _fin_body(p_ref, den_ref, r2_ref, o_ref):
    sump = p_ref[0] + p_ref[1]
    rowd = _iota2((8, 128), 0)
    cold = _iota2((8, 128), 1)
    bcast = (cold // TBW == rowd).astype(jnp.float32)
    denb = jnp.dot(den_ref[...], bcast, preferred_element_type=jnp.float32)
    z = sump / denb + r2_ref[...]
    m = jnp.max(z, axis=1, keepdims=True)
    e = jnp.exp(z - m)
    rg = _iota2((128, 128), 0)
    cg = _iota2((128, 128), 1)
    gmat = (rg // TBW == cg // TBW).astype(jnp.float32)
    gs = jnp.dot(e, gmat, preferred_element_type=jnp.float32)
    o_ref[...] = z - m - jnp.log(gs)


_fin = pl.pallas_call(
    _fin_body,
    grid=(GRID,),
    in_specs=[pl.BlockSpec((NC, BLK, 128), lambda i: (0, i, 0)),
              pl.BlockSpec((BLK, 8), lambda i: (i, 0)),
              pl.BlockSpec((BLK, 128), lambda i: (i, 0))],
    out_specs=pl.BlockSpec((BLK, 128), lambda i: (i, 0)),
    out_shape=jax.ShapeDtypeStruct((N_PACK, 128), jnp.float32),
)


def kernel(x, edge_index, W1l, W1r, b1, W2l, W2r, b2):
    ei = edge_index.astype(jnp.int32)
    ei3 = jnp.pad(ei, ((0, 0), (0, E_PAD - E)),
                  constant_values=N).reshape(2, NCHUNKS, CHUNK)
    x_pad = jnp.pad(x, ((0, N_PAD - N), (0, 0))).reshape(N_PACK, 8 * D)
    wl1 = jnp.pad(W1l, ((0, 0), (0, TBW - HID)))
    b1b = jnp.tile(b1, 8).reshape(1, 64)
    b2b = jnp.tile(b2, 8).reshape(1, 128)
    T1p, R1p = _pre(x_pad, wl1, W1r, b1b)
    P1 = _edge_agg(T1p.reshape(N_PAD, TBW), ei3)
    T2p, R2p, denp = _mid(P1.reshape(NC, N_PACK, 128), R1p, W2l, W2r, b2b)
    P2 = _edge_agg(T2p.reshape(N_PAD, TBW), ei3)
    out = _fin(P2.reshape(NC, N_PACK, 128), denp, R2p)
    return out.reshape(N_PAD, TBW)[:N]


# speedup vs baseline: 1.0829x; 1.0829x over previous
import functools

import jax
import jax.numpy as jnp
from jax import lax
from jax.experimental import pallas as pl
from jax.experimental.pallas import tpu as pltpu
from jax.experimental.pallas import tpu_sc as plsc

N = 10000
D = 128
HID = 8
NCLS = 16
TBW = 16
N_PAD = 10240
N_PACK = N_PAD // 8
E = 320000
NC = 2
NS = 16
CHUNK = 128
CPT0 = 144
CPT1 = 16
NCHUNKS = NS * (CPT0 + CPT1)
E_PAD = NCHUNKS * CHUNK
ROWS_PT = N_PAD // NS

_mesh = plsc.VectorSubcoreMesh(core_axis_name="c", subcore_axis_name="s")


@functools.partial(
    pl.kernel,
    out_type=jax.ShapeDtypeStruct((NC, N_PAD, TBW), jnp.float32),
    mesh=_mesh,
    compiler_params=pltpu.CompilerParams(use_tc_tiling_on_sc=False),
    scratch_types=[
        pltpu.VMEM((CPT0, CHUNK), jnp.int32),
        pltpu.VMEM((CPT0, CHUNK), jnp.int32),
        [pltpu.VMEM((CHUNK, TBW), jnp.float32)] * 8,
        pltpu.VMEM_SHARED((N_PAD, TBW), jnp.float32),
        [pltpu.SemaphoreType.DMA] * 4,
    ],
)
def _edge_agg(table_hbm, edges_hbm, out_hbm,
              src_v, dst_v, rows, acc, sems):
    c = lax.axis_index("c")
    s = lax.axis_index("s")
    row0 = s * ROWS_PT
    @pl.loop(0, CHUNK)
    def _(i):
        rows[0][i] = jnp.zeros((TBW,), jnp.float32)

    for t in range(ROWS_PT // CHUNK):
        pltpu.sync_copy(rows[0], acc.at[pl.ds(row0 + t * CHUNK, CHUNK)])

    K = 4
    bufs = (rows[:K], rows[K:])
    gsem = (sems[0], sems[1])
    ssem = (sems[2], sems[3])

    def _fire_g(g, j):
        for k in range(K):
            pltpu.async_copy(table_hbm.at[src_v.at[j + k]], bufs[g][k],
                             gsem[g])

    def _drain_g(g):
        for k in range(K):
            pltpu.make_async_copy(table_hbm.at[src_v.at[0]], bufs[g][k],
                                  gsem[g]).wait()

    def _fire_s(g, j):
        for k in range(K):
            pltpu.async_copy(bufs[g][k], acc.at[dst_v.at[j + k]], ssem[g],
                             add=True)

    def _drain_s(g):
        for k in range(K):
            pltpu.make_async_copy(bufs[g][k], acc.at[dst_v.at[0]],
                                  ssem[g]).wait()

    def _run(base, cpt):
        pltpu.sync_copy(edges_hbm.at[0, pl.ds(base, cpt)],
                        src_v.at[pl.ds(0, cpt)])
        pltpu.sync_copy(edges_hbm.at[1, pl.ds(base, cpt)],
                        dst_v.at[pl.ds(0, cpt)])
        plsc.subcore_barrier()

        _fire_g(0, 0)

        @pl.loop(0, cpt - 2 * K, step=2 * K)
        def _(j):
            _fire_g(1, j + K)
            _drain_g(0)
            _fire_s(0, j)
            _drain_s(0)
            _fire_g(0, j + 2 * K)
            _drain_g(1)
            _fire_s(1, j + K)
            _drain_s(1)

        _fire_g(1, cpt - K)
        _drain_g(0)
        _fire_s(0, cpt - 2 * K)
        _drain_s(0)
        _drain_g(1)
        _fire_s(1, cpt - K)
        _drain_s(1)

    @pl.when(c == 0)
    def _():
        _run(s * CPT0, CPT0)

    @pl.when(c != 0)
    def _():
        _run(NS * CPT0 + s * CPT1, CPT1)

    plsc.subcore_barrier()
    pltpu.sync_copy(acc.at[pl.ds(row0, ROWS_PT)],
                    out_hbm.at[c, pl.ds(row0, ROWS_PT)])


BLK = 640
GRID = N_PACK // BLK


def _iota2(shape, dim):
    return lax.broadcasted_iota(jnp.int32, shape, dim)


def _pre_body(x_ref, wl_ref, wr_ref, b1_ref, t1_ref, r1_ref):
    xb = x_ref[...]
    wl = wl_ref[...]
    wr = wr_ref[...]
    t = jnp.concatenate(
        [jnp.dot(xb[:, i * D:(i + 1) * D], wl,
                 preferred_element_type=jnp.float32) for i in range(8)],
        axis=1)
    col = _iota2((BLK, 128), 1)
    t1_ref[...] = jnp.where(col % TBW == HID, t + 1.0, t)
    r = jnp.concatenate(
        [jnp.dot(xb[:, i * D:(i + 1) * D], wr,
                 preferred_element_type=jnp.float32) for i in range(8)],
        axis=1)
    r1_ref[...] = r + b1_ref[...]


_pre = pl.pallas_call(
    _pre_body,
    grid=(GRID,),
    in_specs=[pl.BlockSpec((BLK, 8 * D), lambda i: (i, 0)),
              pl.BlockSpec((D, TBW), lambda i: (0, 0)),
              pl.BlockSpec((D, HID), lambda i: (0, 0)),
              pl.BlockSpec((1, 64), lambda i: (0, 0))],
    out_specs=[pl.BlockSpec((BLK, 128), lambda i: (i, 0)),
               pl.BlockSpec((BLK, 64), lambda i: (i, 0))],
    out_shape=[jax.ShapeDtypeStruct((N_PACK, 128), jnp.float32),
               jax.ShapeDtypeStruct((N_PACK, 64), jnp.float32)],
)


def _mid_body(p_ref, r1_ref, wl_ref, wr_ref, b2_ref, t2_ref, r2_ref, den_ref):
    ssum = p_ref[0] + p_ref[1]
    row = _iota2((128, 64), 0)
    colc = _iota2((128, 64), 1)
    sel_sum = ((colc // HID) * TBW + colc % HID == row).astype(jnp.float32)
    sel_cnt = ((colc // HID) * TBW + HID == row).astype(jnp.float32)
    sums = jnp.dot(ssum, sel_sum, preferred_element_type=jnp.float32)
    dens = jnp.maximum(
        jnp.dot(ssum, sel_cnt, preferred_element_type=jnp.float32), 1.0)
    h = sums / dens + r1_ref[...]
    h = jnp.where(h > 0, h, jnp.exp(jnp.minimum(h, 0.0)) - 1.0)
    wl = wl_ref[...]
    wr = wr_ref[...]
    t2_ref[...] = jnp.concatenate(
        [jnp.dot(h[:, i * HID:(i + 1) * HID], wl,
                 preferred_element_type=jnp.float32) for i in range(8)],
        axis=1)
    r2 = jnp.concatenate(
        [jnp.dot(h[:, i * HID:(i + 1) * HID], wr,
                 preferred_element_type=jnp.float32) for i in range(8)],
        axis=1)
    r2_ref[...] = r2 + b2_ref[...]
    rowc = _iota2((128, 8), 0)
    col8 = _iota2((128, 8), 1)
    sel_den = (col8 * TBW + HID == rowc).astype(jnp.float32)
    den_ref[...] = jnp.maximum(
        jnp.dot(ssum, sel_den, preferred_element_type=jnp.float32), 1.0)


_mid = pl.pallas_call(
    _mid_body,
    grid=(GRID,),
    in_specs=[pl.BlockSpec((NC, BLK, 128), lambda i: (0, i, 0)),
              pl.BlockSpec((BLK, 64), lambda i: (i, 0)),
              pl.BlockSpec((HID, NCLS), lambda i: (0, 0)),
              pl.BlockSpec((HID, NCLS), lambda i: (0, 0)),
              pl.BlockSpec((1, 128), lambda i: (0, 0))],
    out_specs=[pl.BlockSpec((BLK, 128), lambda i: (i, 0)),
               pl.BlockSpec((BLK, 128), lambda i: (i, 0)),
               pl.BlockSpec((BLK, 8), lambda i: (i, 0))],
    out_shape=[jax.ShapeDtypeStruct((N_PACK, 128), jnp.float32),
               jax.ShapeDtypeStruct((N_PACK, 128), jnp.float32),
               jax.ShapeDtypeStruct((N_PACK, 8), jnp.float32)],
)


def _fin_body(p_ref, den_ref, r2_ref, o_ref):
    sump = p_ref[0] + p_ref[1]
    rowd = _iota2((8, 128), 0)
    cold = _iota2((8, 128), 1)
    bcast = (cold // TBW == rowd).astype(jnp.float32)
    denb = jnp.dot(den_ref[...], bcast, preferred_element_type=jnp.float32)
    z = sump / denb + r2_ref[...]
    m = jnp.max(z, axis=1, keepdims=True)
    e = jnp.exp(z - m)
    rg = _iota2((128, 128), 0)
    cg = _iota2((128, 128), 1)
    gmat = (rg // TBW == cg // TBW).astype(jnp.float32)
    gs = jnp.dot(e, gmat, preferred_element_type=jnp.float32)
    o_ref[...] = z - m - jnp.log(gs)


_fin = pl.pallas_call(
    _fin_body,
    grid=(GRID,),
    in_specs=[pl.BlockSpec((NC, BLK, 128), lambda i: (0, i, 0)),
              pl.BlockSpec((BLK, 8), lambda i: (i, 0)),
              pl.BlockSpec((BLK, 128), lambda i: (i, 0))],
    out_specs=pl.BlockSpec((BLK, 128), lambda i: (i, 0)),
    out_shape=jax.ShapeDtypeStruct((N_PACK, 128), jnp.float32),
)


def kernel(x, edge_index, W1l, W1r, b1, W2l, W2r, b2):
    ei = edge_index.astype(jnp.int32)
    ei3 = jnp.pad(ei, ((0, 0), (0, E_PAD - E)),
                  constant_values=N).reshape(2, NCHUNKS, CHUNK)
    x_pad = jnp.pad(x, ((0, N_PAD - N), (0, 0))).reshape(N_PACK, 8 * D)
    wl1 = jnp.pad(W1l, ((0, 0), (0, TBW - HID)))
    b1b = jnp.tile(b1, 8).reshape(1, 64)
    b2b = jnp.tile(b2, 8).reshape(1, 128)
    T1p, R1p = _pre(x_pad, wl1, W1r, b1b)
    P1 = _edge_agg(T1p.reshape(N_PAD, TBW), ei3)
    T2p, R2p, denp = _mid(P1.reshape(NC, N_PACK, 128), R1p, W2l, W2r, b2b)
    P2 = _edge_agg(T2p.reshape(N_PAD, TBW), ei3)
    out = _fin(P2.reshape(NC, N_PACK, 128), denp, R2p)
    return out.reshape(N_PAD, TBW)[:N]
